# baseline (speedup 1.0000x reference)
import jax
import jax.numpy as jnp
from jax.experimental import pallas as pl
from jax.experimental.pallas import tpu as pltpu

_NUM_BINS = 64
_SIGMA = 0.5
_EPS = 1e-10
_P = 16384


def _soft_weights_t(ref):
    p = ref.shape[-1]
    v = jnp.clip(ref[0, 0], 0.0, 1.0)
    rowi = jax.lax.broadcasted_iota(jnp.int32, (_NUM_BINS, p), 0)
    rowf = rowi.astype(jnp.float32)
    d = v - rowf * (1.0 / (_NUM_BINS - 1))
    e = jnp.exp(-2.0 * d * d)
    s = jnp.sum(e, axis=0, keepdims=True)
    return e * (1.0 / (s + _EPS))


def _accum_kernel(x_ref, y_ref, g_ref):
    nc = pl.program_id(1)

    @pl.when(nc == 0)
    def _():
        g_ref[...] = jnp.zeros_like(g_ref)

    a = _soft_weights_t(x_ref)
    b = _soft_weights_t(y_ref)
    g_ref[0] += jax.lax.dot_general(
        a, b, (((1,), (1,)), ((), ())), preferred_element_type=jnp.float32
    )


def _marginal_hist(x):
    centers = jnp.linspace(0.0, 1.0, _NUM_BINS, dtype=x.dtype)
    diff = x[:, :, None] - centers[None, None, :]
    w = jnp.exp(-0.5 * (diff / _SIGMA) ** 2)
    w = w / (jnp.sum(w, axis=-1, keepdims=True) + _EPS)
    return jnp.sum(w, axis=1)


def kernel(fixed, moving):
    b = fixed.shape[0]
    if fixed.shape[1] == 3:
        fixed = (0.299 * fixed[:, 0] + 0.587 * fixed[:, 1]
                 + 0.114 * fixed[:, 2])[:, None]
    if moving.shape[1] == 3:
        moving = (0.299 * moving[:, 0] + 0.587 * moving[:, 1]
                  + 0.114 * moving[:, 2])[:, None]
    n = fixed.size // b
    nc = n // _P
    xr = fixed.reshape(b, nc, 1, _P)
    yr = moving.reshape(b, nc, 1, _P)

    joint = pl.pallas_call(
        _accum_kernel,
        out_shape=jax.ShapeDtypeStruct((b, _NUM_BINS, _NUM_BINS), jnp.float32),
        grid=(b, nc),
        in_specs=[
            pl.BlockSpec((1, 1, 1, _P), lambda i, j: (i, j, 0, 0)),
            pl.BlockSpec((1, 1, 1, _P), lambda i, j: (i, j, 0, 0)),
        ],
        out_specs=pl.BlockSpec((1, _NUM_BINS, _NUM_BINS), lambda i, j: (i, 0, 0)),
        compiler_params=pltpu.CompilerParams(
            dimension_semantics=("parallel", "arbitrary"),
        ),
        name="mi_gram_accum",
    )(xr, yr)

    fx = jnp.clip(fixed.reshape(b, -1), 0.0, 1.0)
    mv = jnp.clip(moving.reshape(b, -1), 0.0, 1.0)
    hist_x = _marginal_hist(fx)
    hist_x = hist_x / (jnp.sum(hist_x, axis=-1, keepdims=True) + _EPS)
    hist_y = _marginal_hist(mv)
    hist_y = hist_y / (jnp.sum(hist_y, axis=-1, keepdims=True) + _EPS)
    joint = joint / (jnp.sum(joint, axis=(-1, -2), keepdims=True) + _EPS)

    def _ent(p):
        p = p + _EPS
        return -jnp.sum(p * jnp.log(p), axis=-1)

    mi = _ent(hist_x) + _ent(hist_y) - _ent(joint.reshape(b, -1))
    return -jnp.mean(mi)


# speedup vs baseline: 1.0365x; 1.0365x over previous
import jax
import jax.numpy as jnp
from jax.experimental import pallas as pl
from jax.experimental.pallas import tpu as pltpu

_NUM_BINS = 64
_SIGMA = 0.5
_EPS = 1e-10
_P = 16384


def _unnorm_weights_t(ref):
    p = ref.shape[-1]
    v = jnp.clip(ref[0, 0], 0.0, 1.0)
    rowi = jax.lax.broadcasted_iota(jnp.int32, (_NUM_BINS, p), 0)
    rowf = rowi.astype(jnp.float32)
    d = v - rowf * (1.0 / (_NUM_BINS - 1))
    e = jnp.exp2((-2.8853900817779268 * d) * d)
    s = jnp.sum(e, axis=0, keepdims=True)
    return e, s


def _accum_kernel(x_ref, y_ref, g_ref):
    nc = pl.program_id(1)

    @pl.when(nc == 0)
    def _():
        g_ref[...] = jnp.zeros_like(g_ref)

    ex, sx = _unnorm_weights_t(x_ref)
    ey, sy = _unnorm_weights_t(y_ref)
    m = (1.0 / (sx + _EPS)) * (1.0 / (sy + _EPS))
    a = ex * m
    g_ref[0] += jax.lax.dot_general(
        a, ey, (((1,), (1,)), ((), ())), preferred_element_type=jnp.float32
    )


def _marginal_hist(x):
    centers = jnp.linspace(0.0, 1.0, _NUM_BINS, dtype=x.dtype)
    diff = x[:, :, None] - centers[None, None, :]
    w = jnp.exp(-0.5 * (diff / _SIGMA) ** 2)
    w = w / (jnp.sum(w, axis=-1, keepdims=True) + _EPS)
    return jnp.sum(w, axis=1)


def kernel(fixed, moving):
    b = fixed.shape[0]
    if fixed.shape[1] == 3:
        fixed = (0.299 * fixed[:, 0] + 0.587 * fixed[:, 1]
                 + 0.114 * fixed[:, 2])[:, None]
    if moving.shape[1] == 3:
        moving = (0.299 * moving[:, 0] + 0.587 * moving[:, 1]
                  + 0.114 * moving[:, 2])[:, None]
    n = fixed.size // b
    nc = n // _P
    xr = fixed.reshape(b, nc, 1, _P)
    yr = moving.reshape(b, nc, 1, _P)

    joint = pl.pallas_call(
        _accum_kernel,
        out_shape=jax.ShapeDtypeStruct((b, _NUM_BINS, _NUM_BINS), jnp.float32),
        grid=(b, nc),
        in_specs=[
            pl.BlockSpec((1, 1, 1, _P), lambda i, j: (i, j, 0, 0)),
            pl.BlockSpec((1, 1, 1, _P), lambda i, j: (i, j, 0, 0)),
        ],
        out_specs=pl.BlockSpec((1, _NUM_BINS, _NUM_BINS), lambda i, j: (i, 0, 0)),
        compiler_params=pltpu.CompilerParams(
            dimension_semantics=("parallel", "arbitrary"),
        ),
        name="mi_gram_accum",
    )(xr, yr)

    fx = jnp.clip(fixed.reshape(b, -1), 0.0, 1.0)
    mv = jnp.clip(moving.reshape(b, -1), 0.0, 1.0)
    hist_x = _marginal_hist(fx)
    hist_x = hist_x / (jnp.sum(hist_x, axis=-1, keepdims=True) + _EPS)
    hist_y = _marginal_hist(mv)
    hist_y = hist_y / (jnp.sum(hist_y, axis=-1, keepdims=True) + _EPS)
    joint = joint / (jnp.sum(joint, axis=(-1, -2), keepdims=True) + _EPS)

    def _ent(p):
        p = p + _EPS
        return -jnp.sum(p * jnp.log(p), axis=-1)

    mi = _ent(hist_x) + _ent(hist_y) - _ent(joint.reshape(b, -1))
    return -jnp.mean(mi)
